# Initial kernel scaffold; baseline (speedup 1.0000x reference)
import functools

import jax
import jax.numpy as jnp
from jax import lax
from jax.experimental import pallas as pl
from jax.experimental.pallas import tpu as pltpu
from jax.experimental.pallas import tpu_sc as plsc

N = 10000
D = 128
E = 320000

NC = 2
NS = 16
NW = NC * NS
CH = 128
NCH = 79
EPW = NCH * CH
EP = NW * EPW
NR = 10016
RPT = NR // NS

_mesh = plsc.VectorSubcoreMesh(core_axis_name="c", subcore_axis_name="s")


def _zero_vmem(buf, rows, width):
    per_row = width // 16

    def body(i, _):
        r = i // per_row
        c = (i % per_row) * 16
        buf[r, pl.ds(c, 16)] = jnp.zeros((16,), jnp.float32)
        return 0

    lax.fori_loop(0, rows * per_row, body, 0)


@functools.partial(
    pl.kernel,
    out_type=jax.ShapeDtypeStruct((NC, NR, 16), jnp.float32),
    mesh=_mesh,
    scratch_types=[
        pltpu.VMEM((NCH, CH), jnp.int32),
        pltpu.VMEM((CH, 16), jnp.float32),
        pltpu.VMEM((CH, 16), jnp.float32),
        pltpu.VMEM_SHARED((NR, 16), jnp.float32),
    ],
)
def _deg_kernel(dst_hbm, out_hbm, didx, ones, zeros, acc):
    c = lax.axis_index("c")
    s = lax.axis_index("s")
    wid = c * NS + s

    def fill(i, _):
        ones[i, pl.ds(0, 16)] = jnp.full((16,), 1.0, jnp.float32)
        return 0

    lax.fori_loop(0, CH, fill, 0)
    _zero_vmem(zeros, CH, 16)

    base = s * RPT
    for k in range(4):
        pltpu.sync_copy(zeros, acc.at[pl.ds(base + k * CH, CH)])
    pltpu.sync_copy(zeros.at[pl.ds(0, RPT - 4 * CH)],
                    acc.at[pl.ds(base + 4 * CH, RPT - 4 * CH)])
    plsc.subcore_barrier()

    pltpu.sync_copy(dst_hbm.at[wid], didx)

    def step(j, _):
        pltpu.sync_copy(ones, acc.at[didx.at[j]], add=True)
        return 0

    lax.fori_loop(0, NCH, step, 0)
    plsc.subcore_barrier()

    pltpu.sync_copy(acc.at[pl.ds(base, RPT)],
                    out_hbm.at[c].at[pl.ds(base, RPT)])


@functools.partial(
    pl.kernel,
    out_type=jax.ShapeDtypeStruct((NC, NR, D), jnp.float32),
    mesh=_mesh,
    scratch_types=[
        pltpu.VMEM((NCH, CH), jnp.int32),
        pltpu.VMEM((NCH, CH), jnp.int32),
        pltpu.VMEM((CH, D), jnp.float32),
        pltpu.VMEM((CH, D), jnp.float32),
        pltpu.VMEM_SHARED((NR, D), jnp.float32),
        pltpu.SemaphoreType.DMA,
    ],
)
def _hop_kernel(g_hbm, src_hbm, dst_hbm, out_hbm, sidx, didx, rows, zeros,
                acc, sem):
    c = lax.axis_index("c")
    s = lax.axis_index("s")
    wid = c * NS + s

    _zero_vmem(zeros, CH, D)
    base = s * RPT
    for k in range(4):
        pltpu.sync_copy(zeros, acc.at[pl.ds(base + k * CH, CH)])
    pltpu.sync_copy(zeros.at[pl.ds(0, RPT - 4 * CH)],
                    acc.at[pl.ds(base + 4 * CH, RPT - 4 * CH)])
    plsc.subcore_barrier()

    pltpu.sync_copy(src_hbm.at[wid], sidx)
    pltpu.sync_copy(dst_hbm.at[wid], didx)

    def step(j, _):
        pltpu.async_copy(g_hbm.at[sidx.at[j]], rows, sem).wait()
        pltpu.sync_copy(rows, acc.at[didx.at[j]], add=True)
        return 0

    lax.fori_loop(0, NCH, step, 0)
    plsc.subcore_barrier()

    pltpu.sync_copy(acc.at[pl.ds(base, RPT)],
                    out_hbm.at[c].at[pl.ds(base, RPT)])


def _deg_from_acc(degacc_ref):
    da = degacc_ref[...]
    return da[0, :, 0] + da[1, :, 0] + 1.0


def _tc_g0_body(degacc_ref, x_ref, g0_ref):
    deg = _deg_from_acc(degacc_ref)
    dis = lax.rsqrt(deg)[:N]
    g0_ref[...] = x_ref[...] * dis[:, None]


def _tc_g1_body(degacc_ref, acc_ref, g0_ref, g1_ref):
    deg = _deg_from_acc(degacc_ref)
    inv = (1.0 / deg)[:N]
    h = acc_ref[0, pl.ds(0, N), :] + acc_ref[1, pl.ds(0, N), :] + g0_ref[...]
    g1_ref[...] = h * inv[:, None]


def _tc_out_body(degacc_ref, acc_ref, g1_ref, w_ref, b_ref, out_ref):
    deg = _deg_from_acc(degacc_ref)
    dis = lax.rsqrt(deg)[:N]
    h = (acc_ref[0, pl.ds(0, N), :] + acc_ref[1, pl.ds(0, N), :]
         + g1_ref[...]) * dis[:, None]
    out_ref[...] = (jnp.dot(h, w_ref[...], preferred_element_type=jnp.float32)
                    + b_ref[...][None, :])


def kernel(x, edge_index, W, b):
    src = edge_index[0]
    dst = edge_index[1]
    pad = EP - E
    src_p = jnp.concatenate(
        [src, jnp.zeros((pad,), jnp.int32)]).reshape(NW, NCH, CH)
    dst_p = jnp.concatenate(
        [dst, jnp.full((pad,), N, jnp.int32)]).reshape(NW, NCH, CH)

    degacc = _deg_kernel(dst_p)

    g0 = pl.pallas_call(
        _tc_g0_body,
        out_shape=jax.ShapeDtypeStruct((N, D), jnp.float32),
    )(degacc, x)

    acc1 = _hop_kernel(g0, src_p, dst_p)

    g1 = pl.pallas_call(
        _tc_g1_body,
        out_shape=jax.ShapeDtypeStruct((N, D), jnp.float32),
    )(degacc, acc1, g0)

    acc2 = _hop_kernel(g1, src_p, dst_p)

    out = pl.pallas_call(
        _tc_out_body,
        out_shape=jax.ShapeDtypeStruct((N, D), jnp.float32),
    )(degacc, acc2, g1, W, b)
    return out


# speedup vs baseline: 12.9309x; 12.9309x over previous
import functools

import jax
import jax.numpy as jnp
from jax import lax
from jax.experimental import pallas as pl
from jax.experimental.pallas import tpu as pltpu
from jax.experimental.pallas import tpu_sc as plsc

N = 10000
D = 128
E = 320000

NC = 2
NS = 16
NW = NC * NS
CH = 128
NCH = 79
EPW = NCH * CH
EP = NW * EPW
NR = 10112
RPT = NR // NS

_mesh = plsc.VectorSubcoreMesh(core_axis_name="c", subcore_axis_name="s")


def _zero_vmem(buf, rows, width):
    per_row = width // 16

    def body(i, _):
        r = i // per_row
        c = (i % per_row) * 16
        buf[r, pl.ds(c, 16)] = jnp.zeros((16,), jnp.float32)
        return 0

    lax.fori_loop(0, rows * per_row, body, 0)


@functools.partial(
    pl.kernel,
    out_type=jax.ShapeDtypeStruct((NC, NR, 16), jnp.float32),
    mesh=_mesh,
    scratch_types=[
        pltpu.VMEM((NCH, CH), jnp.int32),
        pltpu.VMEM((CH, 16), jnp.float32),
        pltpu.VMEM((CH, 16), jnp.float32),
        pltpu.VMEM_SHARED((NR, 16), jnp.float32),
    ],
)
def _deg_kernel(dst_hbm, out_hbm, didx, ones, zeros, acc):
    c = lax.axis_index("c")
    s = lax.axis_index("s")
    wid = c * NS + s

    def fill(i, _):
        ones[i, pl.ds(0, 16)] = jnp.full((16,), 1.0, jnp.float32)
        return 0

    lax.fori_loop(0, CH, fill, 0)
    _zero_vmem(zeros, CH, 16)

    base = s * RPT
    for k in range(4):
        pltpu.sync_copy(zeros, acc.at[pl.ds(base + k * CH, CH)])
    pltpu.sync_copy(zeros.at[pl.ds(0, RPT - 4 * CH)],
                    acc.at[pl.ds(base + 4 * CH, RPT - 4 * CH)])
    plsc.subcore_barrier()

    pltpu.sync_copy(dst_hbm.at[wid], didx)

    def step(j, _):
        pltpu.sync_copy(ones, acc.at[didx.at[j]], add=True)
        return 0

    lax.fori_loop(0, NCH, step, 0)
    plsc.subcore_barrier()

    pltpu.sync_copy(acc.at[pl.ds(base, RPT)],
                    out_hbm.at[c].at[pl.ds(base, RPT)])


@functools.partial(
    pl.kernel,
    out_type=jax.ShapeDtypeStruct((NC, NR, D), jnp.float32),
    mesh=_mesh,
    scratch_types=[
        pltpu.VMEM((NCH, CH), jnp.int32),
        pltpu.VMEM((NCH, CH), jnp.int32),
        pltpu.VMEM((CH, D), jnp.float32),
        pltpu.VMEM_SHARED((NR, D), jnp.float32),
        pltpu.SemaphoreType.DMA,
    ],
)
def _hop_kernel(g_hbm, src_hbm, dst_hbm, out_hbm, sidx, didx, rows, acc, sem):
    c = lax.axis_index("c")
    s = lax.axis_index("s")
    wid = c * NS + s

    _zero_vmem(rows, CH, D)
    base = s * RPT
    for k in range(4):
        pltpu.sync_copy(rows, acc.at[pl.ds(base + k * CH, CH)])
    pltpu.sync_copy(rows.at[pl.ds(0, RPT - 4 * CH)],
                    acc.at[pl.ds(base + 4 * CH, RPT - 4 * CH)])
    plsc.subcore_barrier()

    pltpu.sync_copy(src_hbm.at[wid], sidx)
    pltpu.sync_copy(dst_hbm.at[wid], didx)

    def step(j, _):
        pltpu.async_copy(g_hbm.at[sidx.at[j]], rows, sem).wait()
        pltpu.sync_copy(rows, acc.at[didx.at[j]], add=True)
        return 0

    lax.fori_loop(0, NCH, step, 0)
    plsc.subcore_barrier()

    pltpu.sync_copy(acc.at[pl.ds(base, RPT)],
                    out_hbm.at[c].at[pl.ds(base, RPT)])


def _deg_from_acc(degacc_ref):
    da = degacc_ref[...]
    return da[0, :, 0] + da[1, :, 0] + 1.0


def _tc_g0_body(degacc_ref, x_ref, g0_ref):
    deg = _deg_from_acc(degacc_ref)
    dis = lax.rsqrt(deg)[:N]
    g0_ref[...] = x_ref[...] * dis[:, None]


def _tc_g1_body(degacc_ref, acc_ref, g0_ref, g1_ref):
    deg = _deg_from_acc(degacc_ref)
    inv = (1.0 / deg)[:N]
    h = acc_ref[0, pl.ds(0, N), :] + acc_ref[1, pl.ds(0, N), :] + g0_ref[...]
    g1_ref[...] = h * inv[:, None]


def _tc_out_body(degacc_ref, acc_ref, g1_ref, w_ref, b_ref, out_ref):
    deg = _deg_from_acc(degacc_ref)
    dis = lax.rsqrt(deg)[:N]
    h = (acc_ref[0, pl.ds(0, N), :] + acc_ref[1, pl.ds(0, N), :]
         + g1_ref[...]) * dis[:, None]
    out_ref[...] = (jnp.dot(h, w_ref[...], preferred_element_type=jnp.float32)
                    + b_ref[...][None, :])


def kernel(x, edge_index, W, b):
    src = edge_index[0]
    dst = edge_index[1]
    pad = EP - E
    src_p = jnp.concatenate(
        [src, jnp.zeros((pad,), jnp.int32)]).reshape(NW, NCH, CH)
    dst_p = jnp.concatenate(
        [dst, jnp.full((pad,), N, jnp.int32)]).reshape(NW, NCH, CH)

    degacc = _deg_kernel(dst_p)

    g0 = pl.pallas_call(
        _tc_g0_body,
        out_shape=jax.ShapeDtypeStruct((N, D), jnp.float32),
    )(degacc, x)

    acc1 = _hop_kernel(g0, src_p, dst_p)

    g1 = pl.pallas_call(
        _tc_g1_body,
        out_shape=jax.ShapeDtypeStruct((N, D), jnp.float32),
    )(degacc, acc1, g0)

    acc2 = _hop_kernel(g1, src_p, dst_p)

    out = pl.pallas_call(
        _tc_out_body,
        out_shape=jax.ShapeDtypeStruct((N, D), jnp.float32),
    )(degacc, acc2, g1, W, b)
    return out
